# Initial kernel scaffold; baseline (speedup 1.0000x reference)
import jax
import jax.numpy as jnp
from jax.experimental import pallas as pl


def kernel(inputs, targets):
    raise NotImplementedError("write your pallas kernel here")


# speedup vs baseline: 1.8501x; 1.8501x over previous
import jax
import jax.numpy as jnp
from jax.experimental import pallas as pl
from jax.experimental.pallas import tpu as pltpu

_N = 16384
_K = 4915
_ROWS = 128
_COLS = 128


def _topk_mean_body(x_ref, t_ref, out_ref):
    x = x_ref[...]
    t = t_ref[...].astype(jnp.float32)
    z = x * (1.0 - 2.0 * t)

    u = jax.lax.bitcast_convert_type(z, jnp.int32)
    key = jnp.where(u >= 0, u, (~u) ^ jnp.int32(-(2**31)))

    def bit_step(i, acc):
        cand = acc + (jnp.int32(1) << (jnp.int32(31) - i))
        cnt = jnp.sum(jnp.where(key >= cand, 1.0, 0.0))
        return jnp.where(cnt >= jnp.float32(_K), cand, acc)

    kth = jax.lax.fori_loop(0, 32, bit_step, jnp.int32(-(2**31)))

    loss = jnp.maximum(z, 0.0) + jnp.log1p(jnp.exp(-jnp.abs(z)))
    gt = key > kth
    c_gt = jnp.sum(jnp.where(gt, 1.0, 0.0))
    sum_gt = jnp.sum(jnp.where(gt, loss, 0.0))
    loss_at = jnp.max(jnp.where(key == kth, loss, -jnp.inf))
    out_ref[0, 0] = (sum_gt + (jnp.float32(_K) - c_gt) * loss_at) / jnp.float32(_K)


def kernel(inputs, targets):
    x = inputs.reshape(_ROWS, _COLS)
    t = targets.astype(jnp.int32).reshape(_ROWS, _COLS)
    out = pl.pallas_call(
        _topk_mean_body,
        out_shape=jax.ShapeDtypeStruct((1, 1), jnp.float32),
        in_specs=[
            pl.BlockSpec(memory_space=pltpu.VMEM),
            pl.BlockSpec(memory_space=pltpu.VMEM),
        ],
        out_specs=pl.BlockSpec(memory_space=pltpu.SMEM),
    )(x, t)
    return out.reshape(())
